# Initial kernel scaffold; baseline (speedup 1.0000x reference)
import jax
import jax.numpy as jnp
from jax.experimental import pallas as pl


def kernel(h_0, edge_index, edge_index_values, gnn_w, gnn_b, dnn_w, dnn_b):
    raise NotImplementedError("write your pallas kernel here")


# speedup vs baseline: 11.0073x; 11.0073x over previous
import functools

import jax
import jax.numpy as jnp
from jax.experimental import pallas as pl
from jax.experimental.pallas import tpu as pltpu
from jax.experimental.pallas import tpu_sc as plsc

N = 100000
NPN = 40000
NTOP = N - NPN
NG = NPN // 4
NLAYER = 6
F = 16
GF = 64

NCORE = 2
NSUB = 16
NWORK = NCORE * NSUB
CHUNK = 128
EW = 100096
EP = NWORK * EW
NP = 100096
NPS = NP // NSUB
ZROWS = 184


def _sc_spmm_body(table, cols, rows, vals, out, colv, rowv, valv, gbuf, zbuf,
                  acc, sem):
  cid = jax.lax.axis_index("c")
  sid = jax.lax.axis_index("s")
  wid = sid * NCORE + cid

  zero16 = jnp.zeros((16,), jnp.float32)
  for r in range(ZROWS):
    zbuf[r] = zero16

  def zcp(k, c):
    pltpu.sync_copy(zbuf, acc.at[pl.ds(sid * NPS + k * ZROWS, ZROWS)])
    return c

  jax.lax.fori_loop(0, NPS // ZROWS, zcp, 0)
  plsc.subcore_barrier()

  def chunk_body(g, c):
    base = wid * EW + g * CHUNK
    pltpu.sync_copy(cols.at[pl.ds(base, CHUNK)], colv)
    pltpu.sync_copy(rows.at[pl.ds(base, CHUNK)], rowv)
    pltpu.sync_copy(vals.at[pl.ds(base, CHUNK)], valv)
    pltpu.async_copy(table.at[colv], gbuf, sem).wait()

    for eb in range(CHUNK // 16):
      vv = valv[pl.ds(eb * 16, 16)]
      for j in range(16):
        e = eb * 16 + j
        gbuf[e] = gbuf[e] * vv[j]

    pltpu.sync_copy(gbuf, acc.at[rowv], add=True)
    return c

  jax.lax.fori_loop(0, EW // CHUNK, chunk_body, 0)
  plsc.subcore_barrier()
  pltpu.sync_copy(acc.at[pl.ds(sid * NPS, NPS)],
                  out.at[cid, pl.ds(sid * NPS, NPS)])


@functools.lru_cache(maxsize=1)
def _build_sc_spmm():
  return pl.kernel(
      _sc_spmm_body,
      out_type=jax.ShapeDtypeStruct((NCORE, NP, F), jnp.float32),
      mesh=plsc.VectorSubcoreMesh(
          core_axis_name="c", subcore_axis_name="s",
          num_cores=NCORE, num_subcores=NSUB),
      scratch_types=[
          pltpu.VMEM((CHUNK,), jnp.int32),
          pltpu.VMEM((CHUNK,), jnp.int32),
          pltpu.VMEM((CHUNK,), jnp.float32),
          pltpu.VMEM((CHUNK, F), jnp.float32),
          pltpu.VMEM((ZROWS, F), jnp.float32),
          pltpu.VMEM_SHARED((NP, F), jnp.float32),
          pltpu.SemaphoreType.DMA,
      ],
      compiler_params=pltpu.CompilerParams(use_tc_tiling_on_sc=False),
  )


def _sc_spmm(table, cols_p, rows_p, vals_p):
  return _build_sc_spmm()(table, cols_p, rows_p, vals_p)


def _tc_init_body(h0, w, b, o):
  o[...] = h0[...] * w[...] + b[...]


def _tc_top_body(a0, a1, h0, a, bvec, bias, o):
  s = a0[...] + a1[...]
  o[...] = (jnp.dot(s, a[...], preferred_element_type=jnp.float32)
            + h0[...] * bvec[...] + bias[...])


def _tc_path_body(a0, a1, h0p, wd, bd, ablk, em, biasg, o):
  s = a0[...] + a1[...]
  p = jnp.dot(s, wd[...], preferred_element_type=jnp.float32) + bd[...]
  hcat = p + jnp.dot(h0p[...], em[...], preferred_element_type=jnp.float32)
  o[...] = (jnp.dot(hcat, ablk[...], preferred_element_type=jnp.float32)
            + biasg[...])


def _tc_final_body(a0, a1, h0p, wd, bd, em, o):
  s = a0[...] + a1[...]
  p = jnp.dot(s, wd[...], preferred_element_type=jnp.float32) + bd[...]
  o[...] = p + jnp.dot(h0p[...], em[...], preferred_element_type=jnp.float32)


def _row_spec(bm, bn):
  return pl.BlockSpec((bm, bn), lambda i: (i, 0))


def _full_spec(bm, bn):
  return pl.BlockSpec((bm, bn), lambda i: (0, 0))


_BT = NTOP // 10
_BP = NG // 10

_tc_init = pl.pallas_call(
    _tc_init_body,
    grid=(10,),
    in_specs=[_row_spec(N // 10, 1), _full_spec(1, F), _full_spec(1, F)],
    out_specs=_row_spec(N // 10, F),
    out_shape=jax.ShapeDtypeStruct((N, F), jnp.float32),
)

_tc_top = pl.pallas_call(
    _tc_top_body,
    grid=(10,),
    in_specs=[_row_spec(_BT, F), _row_spec(_BT, F), _row_spec(_BT, 1),
              _full_spec(F, F), _full_spec(1, F), _full_spec(1, F)],
    out_specs=_row_spec(_BT, F),
    out_shape=jax.ShapeDtypeStruct((NTOP, F), jnp.float32),
)

_tc_path = pl.pallas_call(
    _tc_path_body,
    grid=(10,),
    in_specs=[_row_spec(_BP, GF), _row_spec(_BP, GF), _row_spec(_BP, 4),
              _full_spec(GF, GF), _full_spec(1, GF), _full_spec(GF, GF),
              _full_spec(4, GF), _full_spec(1, GF)],
    out_specs=_row_spec(_BP, GF),
    out_shape=jax.ShapeDtypeStruct((NG, GF), jnp.float32),
)

_tc_final = pl.pallas_call(
    _tc_final_body,
    grid=(10,),
    in_specs=[_row_spec(_BP, GF), _row_spec(_BP, GF), _row_spec(_BP, 4),
              _full_spec(GF, GF), _full_spec(1, GF), _full_spec(4, GF)],
    out_specs=_row_spec(_BP, GF),
    out_shape=jax.ShapeDtypeStruct((NG, GF), jnp.float32),
)


def _pad_weights(i, gnn_w, gnn_b, dnn_w, dnn_b):
  d = i + 1
  wd = dnn_w[i]
  bd = dnn_b[i]
  wdt = jnp.zeros((GF, GF), jnp.float32)
  for js in range(4):
    for jo in range(4):
      blk = wd.T[js * d:(js + 1) * d, jo * d:(jo + 1) * d]
      wdt = wdt.at[F * js:F * js + d, F * jo:F * jo + d].set(blk)
  bdg = jnp.zeros((1, GF), jnp.float32)
  em = jnp.zeros((4, GF), jnp.float32)
  for j in range(4):
    bdg = bdg.at[0, F * j:F * j + d].set(bd[j * d:(j + 1) * d])
    em = em.at[j, F * j + d].set(1.0)
  if i == NLAYER - 1:
    return wdt, bdg, em

  wg = gnn_w[i + 1]
  bg = gnn_b[i + 1]
  a = jnp.zeros((F, F), jnp.float32).at[:d, :d + 1].set(wg.T[:d, :])
  bvec = jnp.zeros((1, F), jnp.float32).at[0, :d + 1].set(wg[:, d])
  bias = jnp.zeros((1, F), jnp.float32).at[0, :d + 1].set(bg)
  ablk = jnp.zeros((GF, GF), jnp.float32)
  biasg = jnp.zeros((1, GF), jnp.float32)
  for j in range(4):
    ablk = ablk.at[F * j:F * j + d + 1, F * j:F * j + d + 1].set(wg.T)
    biasg = biasg.at[0, F * j:F * j + d + 1].set(bg)
  return wdt, bdg, em, a, bvec, bias, ablk, biasg


def kernel(h_0, edge_index, edge_index_values, gnn_w, gnn_b, dnn_w, dnn_b):
  f32 = jnp.float32
  e = edge_index.shape[1]
  pad = EP - e
  rows_p = jnp.concatenate([edge_index[0], jnp.zeros((pad,), jnp.int32)])
  cols_p = jnp.concatenate([edge_index[1], jnp.zeros((pad,), jnp.int32)])
  vals_p = jnp.concatenate([edge_index_values, jnp.zeros((pad,), f32)])

  h0_top = h_0[:NTOP]
  h0p = h_0[NTOP:].reshape(NG, 4)

  w0 = jnp.zeros((1, F), f32).at[0, 0].set(gnn_w[0][0, 0])
  b0 = jnp.zeros((1, F), f32).at[0, 0].set(gnn_b[0][0])
  table = _tc_init(h_0, w0, b0)

  for i in range(NLAYER):
    acc = _sc_spmm(table, cols_p, rows_p, vals_p)
    a0t, a1t = acc[0, :NTOP], acc[1, :NTOP]
    a0p = acc[0, NTOP:N].reshape(NG, GF)
    a1p = acc[1, NTOP:N].reshape(NG, GF)
    if i < NLAYER - 1:
      wdt, bdg, em, a, bvec, bias, ablk, biasg = _pad_weights(
          i, gnn_w, gnn_b, dnn_w, dnn_b)
      top = _tc_top(a0t, a1t, h0_top, a, bvec, bias)
      pathg = _tc_path(a0p, a1p, h0p, wdt, bdg, ablk, em, biasg)
      table = jnp.concatenate([top, pathg.reshape(NPN, F)], axis=0)
    else:
      wdt, bdg, em = _pad_weights(i, gnn_w, gnn_b, dnn_w, dnn_b)
      outg = _tc_final(a0p, a1p, h0p, wdt, bdg, em)
      return outg.reshape(NPN, F)[:, :NLAYER + 1]


# speedup vs baseline: 30.6567x; 2.7851x over previous
import functools

import jax
import jax.numpy as jnp
from jax.experimental import pallas as pl
from jax.experimental.pallas import tpu as pltpu
from jax.experimental.pallas import tpu_sc as plsc

N = 100000
NPN = 40000
NTOP = N - NPN
NG = NPN // 4
NLAYER = 6
F = 16
GF = 64

NCORE = 2
NSUB = 16
NWORK = NCORE * NSUB
CHUNK = 128
KB = 4
SUP = KB * CHUNK
EW = 100352
EP = NWORK * EW
NSUPS = EW // SUP
EROWS = EP // CHUNK
NP = 100096
NPS = NP // NSUB
ZROWS = 136


def _sc_spmm_body(table, cols, rows, vals, out,
                  colv0, rowv0, valv0, colv1, rowv1, valv1,
                  gbuf0, gbuf1, zbuf, acc, sem_i, sem_g, sem_s):
  cid = jax.lax.axis_index("c")
  sid = jax.lax.axis_index("s")
  wid = sid * NCORE + cid
  wrow = wid * (EW // CHUNK)
  cbufs = (colv0, colv1)
  rbufs = (rowv0, rowv1)
  vbufs = (valv0, valv1)
  gbufs = (gbuf0, gbuf1)

  zero16 = jnp.zeros((16,), jnp.float32)
  for r in range(ZROWS):
    zbuf[r] = zero16

  def zcp(k, c):
    pltpu.sync_copy(zbuf, acc.at[pl.ds(sid * NPS + k * ZROWS, ZROWS)])
    return c

  jax.lax.fori_loop(0, NPS // ZROWS, zcp, 0)
  plsc.subcore_barrier()

  def issue_idx(s, b):
    r0 = wrow + s * KB
    pltpu.async_copy(cols.at[pl.ds(r0, KB)], cbufs[b], sem_i)
    pltpu.async_copy(rows.at[pl.ds(r0, KB)], rbufs[b], sem_i)
    pltpu.async_copy(vals.at[pl.ds(r0, KB)], vbufs[b], sem_i)

  def drain_idx(b):
    pltpu.make_async_copy(cols.at[pl.ds(0, KB)], cbufs[b], sem_i).wait()
    pltpu.make_async_copy(rows.at[pl.ds(0, KB)], rbufs[b], sem_i).wait()
    pltpu.make_async_copy(vals.at[pl.ds(0, KB)], vbufs[b], sem_i).wait()

  def drain_8(gb, sem):
    for k in range(KB):
      pltpu.make_async_copy(table.at[pl.ds(0, CHUNK)],
                            gb.at[pl.ds(k * CHUNK, CHUNK)], sem).wait()

  issue_idx(0, 0)

  def outer(go, c):
    for b in range(2):
      s = go * 2 + b
      drain_idx(b)

      @pl.when(s > 0)
      def _():
        drain_8(gbufs[1 - b], sem_s)

      @pl.when(s < NSUPS - 1)
      def _():
        issue_idx(s + 1, 1 - b)

      gb, vb = gbufs[b], vbufs[b]
      for k in range(KB):
        pltpu.async_copy(table.at[cbufs[b].at[k]],
                         gb.at[pl.ds(k * CHUNK, CHUNK)], sem_g)
      drain_8(gb, sem_g)

      def scale_k(k, cc):
        def scale_eb(eb, cc2):
          vv = vb[k, pl.ds(eb * 16, 16)]
          for j in range(16):
            e = k * CHUNK + eb * 16 + j
            gb[e] = gb[e] * vv[j]
          return cc2

        jax.lax.fori_loop(0, CHUNK // 16, scale_eb, 0)
        return cc

      jax.lax.fori_loop(0, KB, scale_k, 0)

      for k in range(KB):
        pltpu.async_copy(gb.at[pl.ds(k * CHUNK, CHUNK)],
                         acc.at[rbufs[b].at[k]], sem_s, add=True)
    return c

  jax.lax.fori_loop(0, NSUPS // 2, outer, 0)
  drain_8(gbufs[1], sem_s)
  plsc.subcore_barrier()
  pltpu.sync_copy(acc.at[pl.ds(sid * NPS, NPS)],
                  out.at[cid, pl.ds(sid * NPS, NPS)])


@functools.lru_cache(maxsize=1)
def _build_sc_spmm():
  return pl.kernel(
      _sc_spmm_body,
      out_type=jax.ShapeDtypeStruct((NCORE, NP, F), jnp.float32),
      mesh=plsc.VectorSubcoreMesh(
          core_axis_name="c", subcore_axis_name="s",
          num_cores=NCORE, num_subcores=NSUB),
      scratch_types=[
          pltpu.VMEM((KB, CHUNK), jnp.int32),
          pltpu.VMEM((KB, CHUNK), jnp.int32),
          pltpu.VMEM((KB, CHUNK), jnp.float32),
          pltpu.VMEM((KB, CHUNK), jnp.int32),
          pltpu.VMEM((KB, CHUNK), jnp.int32),
          pltpu.VMEM((KB, CHUNK), jnp.float32),
          pltpu.VMEM((SUP, F), jnp.float32),
          pltpu.VMEM((SUP, F), jnp.float32),
          pltpu.VMEM((ZROWS, F), jnp.float32),
          pltpu.VMEM_SHARED((NP, F), jnp.float32),
          pltpu.SemaphoreType.DMA,
          pltpu.SemaphoreType.DMA,
          pltpu.SemaphoreType.DMA,
      ],
      compiler_params=pltpu.CompilerParams(use_tc_tiling_on_sc=False),
  )


def _sc_spmm(table, cols_p, rows_p, vals_p):
  return _build_sc_spmm()(table, cols_p, rows_p, vals_p)


def _tc_init_body(h0, w, b, o):
  o[...] = h0[...] * w[...] + b[...]


def _tc_top_body(a0, a1, h0, a, bvec, bias, o):
  s = a0[...] + a1[...]
  o[...] = (jnp.dot(s, a[...], preferred_element_type=jnp.float32)
            + h0[...] * bvec[...] + bias[...])


def _tc_path_body(a0, a1, h0p, wd, bd, ablk, em, biasg, o):
  s = a0[...] + a1[...]
  p = jnp.dot(s, wd[...], preferred_element_type=jnp.float32) + bd[...]
  hcat = p + jnp.dot(h0p[...], em[...], preferred_element_type=jnp.float32)
  o[...] = (jnp.dot(hcat, ablk[...], preferred_element_type=jnp.float32)
            + biasg[...])


def _tc_final_body(a0, a1, h0p, wd, bd, em, o):
  s = a0[...] + a1[...]
  p = jnp.dot(s, wd[...], preferred_element_type=jnp.float32) + bd[...]
  o[...] = p + jnp.dot(h0p[...], em[...], preferred_element_type=jnp.float32)


def _row_spec(bm, bn):
  return pl.BlockSpec((bm, bn), lambda i: (i, 0))


def _full_spec(bm, bn):
  return pl.BlockSpec((bm, bn), lambda i: (0, 0))


_BT = NTOP // 10
_BP = NG // 10

_tc_init = pl.pallas_call(
    _tc_init_body,
    grid=(10,),
    in_specs=[_row_spec(N // 10, 1), _full_spec(1, F), _full_spec(1, F)],
    out_specs=_row_spec(N // 10, F),
    out_shape=jax.ShapeDtypeStruct((N, F), jnp.float32),
)

_tc_top = pl.pallas_call(
    _tc_top_body,
    grid=(10,),
    in_specs=[_row_spec(_BT, F), _row_spec(_BT, F), _row_spec(_BT, 1),
              _full_spec(F, F), _full_spec(1, F), _full_spec(1, F)],
    out_specs=_row_spec(_BT, F),
    out_shape=jax.ShapeDtypeStruct((NTOP, F), jnp.float32),
)

_tc_path = pl.pallas_call(
    _tc_path_body,
    grid=(10,),
    in_specs=[_row_spec(_BP, GF), _row_spec(_BP, GF), _row_spec(_BP, 4),
              _full_spec(GF, GF), _full_spec(1, GF), _full_spec(GF, GF),
              _full_spec(4, GF), _full_spec(1, GF)],
    out_specs=_row_spec(_BP, GF),
    out_shape=jax.ShapeDtypeStruct((NG, GF), jnp.float32),
)

_tc_final = pl.pallas_call(
    _tc_final_body,
    grid=(10,),
    in_specs=[_row_spec(_BP, GF), _row_spec(_BP, GF), _row_spec(_BP, 4),
              _full_spec(GF, GF), _full_spec(1, GF), _full_spec(4, GF)],
    out_specs=_row_spec(_BP, GF),
    out_shape=jax.ShapeDtypeStruct((NG, GF), jnp.float32),
)


def _pad_weights(i, gnn_w, gnn_b, dnn_w, dnn_b):
  d = i + 1
  wd = dnn_w[i]
  bd = dnn_b[i]
  wdt = jnp.zeros((GF, GF), jnp.float32)
  for js in range(4):
    for jo in range(4):
      blk = wd.T[js * d:(js + 1) * d, jo * d:(jo + 1) * d]
      wdt = wdt.at[F * js:F * js + d, F * jo:F * jo + d].set(blk)
  bdg = jnp.zeros((1, GF), jnp.float32)
  em = jnp.zeros((4, GF), jnp.float32)
  for j in range(4):
    bdg = bdg.at[0, F * j:F * j + d].set(bd[j * d:(j + 1) * d])
    em = em.at[j, F * j + d].set(1.0)
  if i == NLAYER - 1:
    return wdt, bdg, em

  wg = gnn_w[i + 1]
  bg = gnn_b[i + 1]
  a = jnp.zeros((F, F), jnp.float32).at[:d, :d + 1].set(wg.T[:d, :])
  bvec = jnp.zeros((1, F), jnp.float32).at[0, :d + 1].set(wg[:, d])
  bias = jnp.zeros((1, F), jnp.float32).at[0, :d + 1].set(bg)
  ablk = jnp.zeros((GF, GF), jnp.float32)
  biasg = jnp.zeros((1, GF), jnp.float32)
  for j in range(4):
    ablk = ablk.at[F * j:F * j + d + 1, F * j:F * j + d + 1].set(wg.T)
    biasg = biasg.at[0, F * j:F * j + d + 1].set(bg)
  return wdt, bdg, em, a, bvec, bias, ablk, biasg


def kernel(h_0, edge_index, edge_index_values, gnn_w, gnn_b, dnn_w, dnn_b):
  f32 = jnp.float32
  e = edge_index.shape[1]
  pad = EP - e
  rows_p = jnp.concatenate(
      [edge_index[0], jnp.zeros((pad,), jnp.int32)]).reshape(EROWS, CHUNK)
  cols_p = jnp.concatenate(
      [edge_index[1], jnp.zeros((pad,), jnp.int32)]).reshape(EROWS, CHUNK)
  vals_p = jnp.concatenate(
      [edge_index_values, jnp.zeros((pad,), f32)]).reshape(EROWS, CHUNK)

  h0_top = h_0[:NTOP]
  h0p = h_0[NTOP:].reshape(NG, 4)

  w0 = jnp.zeros((1, F), f32).at[0, 0].set(gnn_w[0][0, 0])
  b0 = jnp.zeros((1, F), f32).at[0, 0].set(gnn_b[0][0])
  table = _tc_init(h_0, w0, b0)

  for i in range(NLAYER):
    acc = _sc_spmm(table, cols_p, rows_p, vals_p)
    a0t, a1t = acc[0, :NTOP], acc[1, :NTOP]
    a0p = acc[0, NTOP:N].reshape(NG, GF)
    a1p = acc[1, NTOP:N].reshape(NG, GF)
    if i < NLAYER - 1:
      wdt, bdg, em, a, bvec, bias, ablk, biasg = _pad_weights(
          i, gnn_w, gnn_b, dnn_w, dnn_b)
      top = _tc_top(a0t, a1t, h0_top, a, bvec, bias)
      pathg = _tc_path(a0p, a1p, h0p, wdt, bdg, ablk, em, biasg)
      table = jnp.concatenate([top, pathg.reshape(NPN, F)], axis=0)
    else:
      wdt, bdg, em = _pad_weights(i, gnn_w, gnn_b, dnn_w, dnn_b)
      outg = _tc_final(a0p, a1p, h0p, wdt, bdg, em)
      return outg.reshape(NPN, F)[:, :NLAYER + 1]


# speedup vs baseline: 33.5037x; 1.0929x over previous
import functools

import jax
import jax.numpy as jnp
from jax.experimental import pallas as pl
from jax.experimental.pallas import tpu as pltpu
from jax.experimental.pallas import tpu_sc as plsc

N = 100000
NPN = 40000
NTOP = N - NPN
NG = NPN // 4
NLAYER = 6
F = 16
GF = 64

NCORE = 2
NSUB = 16
NWORK = NCORE * NSUB
CHUNK = 128
KB = 4
SUP = KB * CHUNK
EW = 100352
EP = NWORK * EW
NSUPS = EW // SUP
EROWS = EP // CHUNK
NP = 100096
NPS = NP // NSUB
ZROWS = 136


def _sc_spmm_body(table, cols, rows, vals, out,
                  colv0, rowv0, valv0, colv1, rowv1, valv1,
                  gbuf0, gbuf1, zbuf, acc, sem_i, sem_g, sem_s):
  cid = jax.lax.axis_index("c")
  sid = jax.lax.axis_index("s")
  wid = sid * NCORE + cid
  wrow = wid * (EW // CHUNK)
  cbufs = (colv0, colv1)
  rbufs = (rowv0, rowv1)
  vbufs = (valv0, valv1)
  gbufs = (gbuf0, gbuf1)

  zero16 = jnp.zeros((16,), jnp.float32)
  for r in range(ZROWS):
    zbuf[r] = zero16

  def zcp(k, c):
    pltpu.sync_copy(zbuf, acc.at[pl.ds(sid * NPS + k * ZROWS, ZROWS)])
    return c

  jax.lax.fori_loop(0, NPS // ZROWS, zcp, 0)
  plsc.subcore_barrier()

  def issue_idx(s, b):
    r0 = wrow + s * KB
    pltpu.async_copy(cols.at[pl.ds(r0, KB)], cbufs[b], sem_i)
    pltpu.async_copy(rows.at[pl.ds(r0, KB)], rbufs[b], sem_i)
    pltpu.async_copy(vals.at[pl.ds(r0, KB)], vbufs[b], sem_i)

  def drain_idx(b):
    pltpu.make_async_copy(cols.at[pl.ds(0, KB)], cbufs[b], sem_i).wait()
    pltpu.make_async_copy(rows.at[pl.ds(0, KB)], rbufs[b], sem_i).wait()
    pltpu.make_async_copy(vals.at[pl.ds(0, KB)], vbufs[b], sem_i).wait()

  def drain_8(gb, sem):
    for k in range(KB):
      pltpu.make_async_copy(table.at[pl.ds(0, CHUNK)],
                            gb.at[pl.ds(k * CHUNK, CHUNK)], sem).wait()

  issue_idx(0, 0)

  def outer(go, c):
    for b in range(2):
      s = go * 2 + b
      drain_idx(b)

      @pl.when(s > 0)
      def _():
        drain_8(gbufs[1 - b], sem_s)

      @pl.when(s < NSUPS - 1)
      def _():
        issue_idx(s + 1, 1 - b)

      gb, vb = gbufs[b], vbufs[b]
      for k in range(KB):
        pltpu.async_copy(table.at[cbufs[b].at[k]],
                         gb.at[pl.ds(k * CHUNK, CHUNK)], sem_g)
      drain_8(gb, sem_g)

      def scale_k(k, cc):
        def scale_eb(eb, cc2):
          vv = vb[k, pl.ds(eb * 16, 16)]
          for j in range(16):
            e = k * CHUNK + eb * 16 + j
            gb[e] = gb[e] * vv[j]
          return cc2

        jax.lax.fori_loop(0, CHUNK // 16, scale_eb, 0)
        return cc

      jax.lax.fori_loop(0, KB, scale_k, 0)

      for k in range(KB):
        pltpu.async_copy(gb.at[pl.ds(k * CHUNK, CHUNK)],
                         acc.at[rbufs[b].at[k]], sem_s, add=True)
    return c

  jax.lax.fori_loop(0, NSUPS // 2, outer, 0)
  drain_8(gbufs[1], sem_s)
  plsc.subcore_barrier()
  pltpu.sync_copy(acc.at[pl.ds(sid * NPS, NPS)],
                  out.at[cid, pl.ds(sid * NPS, NPS)])


@functools.lru_cache(maxsize=1)
def _build_sc_spmm():
  return pl.kernel(
      _sc_spmm_body,
      out_type=jax.ShapeDtypeStruct((NCORE, NP, F), jnp.float32),
      mesh=plsc.VectorSubcoreMesh(
          core_axis_name="c", subcore_axis_name="s",
          num_cores=NCORE, num_subcores=NSUB),
      scratch_types=[
          pltpu.VMEM((KB, CHUNK), jnp.int32),
          pltpu.VMEM((KB, CHUNK), jnp.int32),
          pltpu.VMEM((KB, CHUNK), jnp.float32),
          pltpu.VMEM((KB, CHUNK), jnp.int32),
          pltpu.VMEM((KB, CHUNK), jnp.int32),
          pltpu.VMEM((KB, CHUNK), jnp.float32),
          pltpu.VMEM((SUP, F), jnp.float32),
          pltpu.VMEM((SUP, F), jnp.float32),
          pltpu.VMEM((ZROWS, F), jnp.float32),
          pltpu.VMEM_SHARED((NP, F), jnp.float32),
          pltpu.SemaphoreType.DMA,
          pltpu.SemaphoreType.DMA,
          pltpu.SemaphoreType.DMA,
      ],
      compiler_params=pltpu.CompilerParams(use_tc_tiling_on_sc=False),
  )


def _sc_spmm(table, cols_p, rows_p, vals_p):
  return _build_sc_spmm()(table, cols_p, rows_p, vals_p)


_BR = 4000
_NBLK = N // _BR
_PBLK = NTOP // _BR


def _tc_init_body(h0, w, b, o):
  o[...] = h0[...] * w[...] + b[...]


def _path_dnn(s, m, bdr):
  rowmod = jax.lax.broadcasted_iota(jnp.int32, (_BR, F), 0) % 4
  p = jnp.zeros((_BR, F), jnp.float32)
  for j in range(4):
    accj = jnp.zeros((_BR, F), jnp.float32)
    for js in range(4):
      rs = s if j == js else jnp.roll(s, j - js, axis=0)
      accj = accj + jnp.dot(rs, m[j, js], preferred_element_type=jnp.float32)
    accj = accj + bdr[j]
    p = jnp.where(rowmod == j, accj, p)
  return p


def _tc_bound_body(a0, a1, h0, apad, ed, bias, m, bdr, o):
  i = pl.program_id(0)
  s = a0[0] + a1[0]
  hin = h0[...] * ed[...]

  @pl.when(i < _PBLK)
  def _():
    o[...] = (jnp.dot(s + hin, apad[...],
                      preferred_element_type=jnp.float32) + bias[...])

  @pl.when(i >= _PBLK)
  def _():
    p = _path_dnn(s, m, bdr)
    o[...] = (jnp.dot(p + hin, apad[...],
                      preferred_element_type=jnp.float32) + bias[...])


def _tc_final_body(a0, a1, h0, ed, m, bdr, o):
  s = a0[0] + a1[0]
  o[...] = _path_dnn(s, m, bdr) + h0[...] * ed[...]


def _row_spec(bm, bn):
  return pl.BlockSpec((bm, bn), lambda i: (i, 0))


def _full_spec(bm, bn):
  return pl.BlockSpec((bm, bn), lambda i: (0, 0))


def _acc_spec(core):
  return pl.BlockSpec((1, _BR, F), lambda i: (core, i, 0))


def _acc_spec_path(core):
  return pl.BlockSpec((1, _BR, F), lambda i: (core, _PBLK + i, 0))


_tc_init = pl.pallas_call(
    _tc_init_body,
    grid=(10,),
    in_specs=[_row_spec(N // 10, 1), _full_spec(1, F), _full_spec(1, F)],
    out_specs=_row_spec(N // 10, F),
    out_shape=jax.ShapeDtypeStruct((N, F), jnp.float32),
)

_tc_bound = pl.pallas_call(
    _tc_bound_body,
    grid=(_NBLK,),
    in_specs=[_acc_spec(0), _acc_spec(1), _row_spec(_BR, 1),
              _full_spec(F, F), _full_spec(1, F), _full_spec(1, F),
              pl.BlockSpec((4, 4, F, F), lambda i: (0, 0, 0, 0)),
              pl.BlockSpec((4, F), lambda i: (0, 0))],
    out_specs=_row_spec(_BR, F),
    out_shape=jax.ShapeDtypeStruct((N, F), jnp.float32),
)

_tc_final = pl.pallas_call(
    _tc_final_body,
    grid=(NPN // _BR,),
    in_specs=[_acc_spec_path(0), _acc_spec_path(1),
              pl.BlockSpec((_BR, 1), lambda i: (_PBLK + i, 0)),
              _full_spec(1, F),
              pl.BlockSpec((4, 4, F, F), lambda i: (0, 0, 0, 0)),
              pl.BlockSpec((4, F), lambda i: (0, 0))],
    out_specs=_row_spec(_BR, F),
    out_shape=jax.ShapeDtypeStruct((NPN, F), jnp.float32),
)


def _pad_weights(i, gnn_w, gnn_b, dnn_w, dnn_b):
  d = i + 1
  wd = dnn_w[i]
  bd = dnn_b[i]
  m = jnp.zeros((4, 4, F, F), jnp.float32)
  bdr = jnp.zeros((4, F), jnp.float32)
  for j in range(4):
    bdr = bdr.at[j, :d].set(bd[j * d:(j + 1) * d])
    for js in range(4):
      blk = wd[j * d:(j + 1) * d, js * d:(js + 1) * d]
      m = m.at[j, js, :d, :d].set(blk.T)
  ed = jnp.zeros((1, F), jnp.float32).at[0, d].set(1.0)
  if i == NLAYER - 1:
    return m, bdr, ed

  wg = gnn_w[i + 1]
  bg = gnn_b[i + 1]
  apad = jnp.zeros((F, F), jnp.float32).at[:d + 1, :d + 1].set(wg.T)
  bias = jnp.zeros((1, F), jnp.float32).at[0, :d + 1].set(bg)
  return m, bdr, ed, apad, bias


def kernel(h_0, edge_index, edge_index_values, gnn_w, gnn_b, dnn_w, dnn_b):
  f32 = jnp.float32
  e = edge_index.shape[1]
  pad = EP - e
  rows_p = jnp.concatenate(
      [edge_index[0], jnp.zeros((pad,), jnp.int32)]).reshape(EROWS, CHUNK)
  cols_p = jnp.concatenate(
      [edge_index[1], jnp.zeros((pad,), jnp.int32)]).reshape(EROWS, CHUNK)
  vals_p = jnp.concatenate(
      [edge_index_values, jnp.zeros((pad,), f32)]).reshape(EROWS, CHUNK)

  w0 = jnp.zeros((1, F), f32).at[0, 0].set(gnn_w[0][0, 0])
  b0 = jnp.zeros((1, F), f32).at[0, 0].set(gnn_b[0][0])
  table = _tc_init(h_0, w0, b0)

  for i in range(NLAYER):
    acc = _sc_spmm(table, cols_p, rows_p, vals_p)
    if i < NLAYER - 1:
      m, bdr, ed, apad, bias = _pad_weights(i, gnn_w, gnn_b, dnn_w, dnn_b)
      table = _tc_bound(acc, acc, h_0, apad, ed, bias, m, bdr)
    else:
      m, bdr, ed = _pad_weights(i, gnn_w, gnn_b, dnn_w, dnn_b)
      outp = _tc_final(acc, acc, h_0, ed, m, bdr)
      return outp[:, :NLAYER + 1]


# speedup vs baseline: 45.7873x; 1.3666x over previous
import functools

import jax
import jax.numpy as jnp
from jax.experimental import pallas as pl
from jax.experimental.pallas import tpu as pltpu
from jax.experimental.pallas import tpu_sc as plsc

N = 100000
NPN = 40000
NTOP = N - NPN
NG = NPN // 4
NLAYER = 6
F = 16
GF = 64

NCORE = 2
NSUB = 16
NWORK = NCORE * NSUB
CHUNK = 128
KB = 4
SUP = KB * CHUNK
EW = 100352
EP = NWORK * EW
NSUPS = EW // SUP
EROWS = EP // CHUNK
NP = 100096
NPS = NP // NSUB
ZROWS = 136


def _sc_spmm_body(table, cols, rows, vals, out,
                  colv0, rowv0, valv0, colv1, rowv1, valv1,
                  gbuf0, gbuf1, zbuf, acc, sem_i, sem_g, sem_s):
  cid = jax.lax.axis_index("c")
  sid = jax.lax.axis_index("s")
  wid = sid * NCORE + cid
  wrow = wid * (EW // CHUNK)
  cbufs = (colv0, colv1)
  rbufs = (rowv0, rowv1)
  vbufs = (valv0, valv1)
  gbufs = (gbuf0, gbuf1)

  zero16 = jnp.zeros((16,), jnp.float32)
  for r in range(ZROWS):
    zbuf[r] = zero16

  def zcp(k, c):
    pltpu.sync_copy(zbuf, acc.at[pl.ds(sid * NPS + k * ZROWS, ZROWS)])
    return c

  jax.lax.fori_loop(0, NPS // ZROWS, zcp, 0)
  plsc.subcore_barrier()

  def issue_idx(s, b):
    r0 = wrow + s * KB
    pltpu.async_copy(cols.at[pl.ds(r0, KB)], cbufs[b], sem_i)
    pltpu.async_copy(rows.at[pl.ds(r0, KB)], rbufs[b], sem_i)
    pltpu.async_copy(vals.at[pl.ds(r0, KB)], vbufs[b], sem_i)

  def drain_idx(b):
    pltpu.make_async_copy(cols.at[pl.ds(0, KB)], cbufs[b], sem_i).wait()
    pltpu.make_async_copy(rows.at[pl.ds(0, KB)], rbufs[b], sem_i).wait()
    pltpu.make_async_copy(vals.at[pl.ds(0, KB)], vbufs[b], sem_i).wait()

  def drain_8(gb, sem):
    for k in range(KB):
      pltpu.make_async_copy(table.at[pl.ds(0, CHUNK)],
                            gb.at[pl.ds(k * CHUNK, CHUNK)], sem).wait()

  issue_idx(0, 0)

  def outer(go, c):
    for b in range(2):
      s = go * 2 + b
      drain_idx(b)

      @pl.when(s > 0)
      def _():
        drain_8(gbufs[1 - b], sem_s)

      @pl.when(s < NSUPS - 1)
      def _():
        issue_idx(s + 1, 1 - b)

      gb, vb = gbufs[b], vbufs[b]
      for k in range(KB):
        pltpu.async_copy(table.at[cbufs[b].at[k]],
                         gb.at[pl.ds(k * CHUNK, CHUNK)], sem_g)
      drain_8(gb, sem_g)

      def scale_k(k, cc):
        def scale_eb(eb, cc2):
          vv = vb[k, pl.ds(eb * 16, 16)]
          for j in range(16):
            e = k * CHUNK + eb * 16 + j
            gb[e] = gb[e] * vv[j]
          return cc2

        jax.lax.fori_loop(0, CHUNK // 16, scale_eb, 0)
        return cc

      jax.lax.fori_loop(0, KB, scale_k, 0)

      for k in range(KB):
        pltpu.async_copy(gb.at[pl.ds(k * CHUNK, CHUNK)],
                         acc.at[rbufs[b].at[k]], sem_s, add=True)
    return c

  jax.lax.fori_loop(0, NSUPS // 2, outer, 0)
  drain_8(gbufs[1], sem_s)
  plsc.subcore_barrier()
  pltpu.sync_copy(acc.at[pl.ds(sid * NPS, NPS)],
                  out.at[cid, pl.ds(sid * NPS, NPS)])


@functools.lru_cache(maxsize=1)
def _build_sc_spmm():
  return pl.kernel(
      _sc_spmm_body,
      out_type=jax.ShapeDtypeStruct((NCORE, NP, F), jnp.float32),
      mesh=plsc.VectorSubcoreMesh(
          core_axis_name="c", subcore_axis_name="s",
          num_cores=NCORE, num_subcores=NSUB),
      scratch_types=[
          pltpu.VMEM((KB, CHUNK), jnp.int32),
          pltpu.VMEM((KB, CHUNK), jnp.int32),
          pltpu.VMEM((KB, CHUNK), jnp.float32),
          pltpu.VMEM((KB, CHUNK), jnp.int32),
          pltpu.VMEM((KB, CHUNK), jnp.int32),
          pltpu.VMEM((KB, CHUNK), jnp.float32),
          pltpu.VMEM((SUP, F), jnp.float32),
          pltpu.VMEM((SUP, F), jnp.float32),
          pltpu.VMEM((ZROWS, F), jnp.float32),
          pltpu.VMEM_SHARED((NP, F), jnp.float32),
          pltpu.SemaphoreType.DMA,
          pltpu.SemaphoreType.DMA,
          pltpu.SemaphoreType.DMA,
      ],
      compiler_params=pltpu.CompilerParams(use_tc_tiling_on_sc=False),
  )


def _sc_spmm(table, cols_p, rows_p, vals_p):
  return _build_sc_spmm()(table, cols_p, rows_p, vals_p)


PK = 8
NPPK = NP // PK
_PROW = NTOP // PK


def _tc_init_body(h0, w0e, b0, o):
  o[...] = jnp.dot(h0[...], w0e[...], preferred_element_type=jnp.float32) + b0[...]


def _tc_bound_body(acc, h0, wd128, bd128, e8, wtop, bias, o):
  s = acc[0] + acc[1]
  hin = jnp.dot(h0[...], e8[...], preferred_element_type=jnp.float32)
  top = s + hin
  path = (jnp.dot(s, wd128[...], preferred_element_type=jnp.float32)
          + bd128[...] + hin)
  rowi = jax.lax.broadcasted_iota(jnp.int32, (NPPK, 128), 0)
  hn = jnp.where(rowi < _PROW, top, path)
  o[...] = jnp.dot(hn, wtop[...], preferred_element_type=jnp.float32) + bias[...]


def _tc_final_body(acc, h0, wd128, bd128, e8, o):
  s = acc[0] + acc[1]
  o[...] = (jnp.dot(s, wd128[...], preferred_element_type=jnp.float32)
            + bd128[...]
            + jnp.dot(h0[...], e8[...], preferred_element_type=jnp.float32))


_tc_init = pl.pallas_call(
    _tc_init_body,
    out_shape=jax.ShapeDtypeStruct((NPPK, 128), jnp.float32),
)

_tc_bound = pl.pallas_call(
    _tc_bound_body,
    out_shape=jax.ShapeDtypeStruct((NPPK, 128), jnp.float32),
)

_tc_final = pl.pallas_call(
    _tc_final_body,
    out_shape=jax.ShapeDtypeStruct((NPPK, 128), jnp.float32),
)


def _pad_weights(i, gnn_w, gnn_b, dnn_w, dnn_b):
  d = i + 1
  wd = dnn_w[i]
  bd = dnn_b[i]
  wdt64 = jnp.zeros((GF, GF), jnp.float32)
  bd64 = jnp.zeros((GF,), jnp.float32)
  for j in range(4):
    bd64 = bd64.at[F * j:F * j + d].set(bd[j * d:(j + 1) * d])
    for js in range(4):
      blk = wd[j * d:(j + 1) * d, js * d:(js + 1) * d]
      wdt64 = wdt64.at[F * js:F * js + d, F * j:F * j + d].set(blk.T)
  wd128 = jnp.kron(jnp.eye(2, dtype=jnp.float32), wdt64)
  bd128 = jnp.tile(bd64, 2)[None, :]
  e8 = jnp.zeros((PK, 128), jnp.float32)
  for j in range(PK):
    e8 = e8.at[j, F * j + d].set(1.0)
  if i == NLAYER - 1:
    return wd128, bd128, e8

  wg = gnn_w[i + 1]
  bg = gnn_b[i + 1]
  apad = jnp.zeros((F, F), jnp.float32).at[:d + 1, :d + 1].set(wg.T)
  bias16 = jnp.zeros((F,), jnp.float32).at[:d + 1].set(bg)
  wtop = jnp.kron(jnp.eye(PK, dtype=jnp.float32), apad)
  bias = jnp.tile(bias16, PK)[None, :]
  return wd128, bd128, e8, wtop, bias


def kernel(h_0, edge_index, edge_index_values, gnn_w, gnn_b, dnn_w, dnn_b):
  f32 = jnp.float32
  e = edge_index.shape[1]
  pad = EP - e
  rows_p = jnp.concatenate(
      [edge_index[0], jnp.zeros((pad,), jnp.int32)]).reshape(EROWS, CHUNK)
  cols_p = jnp.concatenate(
      [edge_index[1], jnp.zeros((pad,), jnp.int32)]).reshape(EROWS, CHUNK)
  vals_p = jnp.concatenate(
      [edge_index_values, jnp.zeros((pad,), f32)]).reshape(EROWS, CHUNK)

  h0p8 = jnp.concatenate(
      [h_0, jnp.zeros((NP - N, 1), f32)]).reshape(NPPK, PK)

  w0e = jnp.zeros((PK, 128), f32)
  for j in range(PK):
    w0e = w0e.at[j, F * j].set(gnn_w[0][0, 0])
  b0 = jnp.tile(jnp.zeros((F,), f32).at[0].set(gnn_b[0][0]), PK)[None, :]
  tablep = _tc_init(h0p8, w0e, b0)

  for i in range(NLAYER):
    acc = _sc_spmm(tablep.reshape(NP, F), cols_p, rows_p, vals_p)
    accp = acc.reshape(NCORE, NPPK, 128)
    if i < NLAYER - 1:
      wd128, bd128, e8, wtop, bias = _pad_weights(i, gnn_w, gnn_b, dnn_w, dnn_b)
      tablep = _tc_bound(accp, h0p8, wd128, bd128, e8, wtop, bias)
    else:
      wd128, bd128, e8 = _pad_weights(i, gnn_w, gnn_b, dnn_w, dnn_b)
      outp = _tc_final(accp, h0p8, wd128, bd128, e8)
      return outp.reshape(NP, F)[NTOP:N, :NLAYER + 1]
